# Initial kernel scaffold; baseline (speedup 1.0000x reference)
import jax
import jax.numpy as jnp
from jax.experimental import pallas as pl


def kernel(actor_ids, role_types, career_features, genre_distribution, actor_table, role_table, cW1, cb1, cW2, cb2, gW, gb, fW1, fb1, fW2, fb2):
    raise NotImplementedError("write your pallas kernel here")


# speedup vs baseline: 1.6240x; 1.6240x over previous
import functools

import jax
import jax.numpy as jnp
from jax import lax
from jax.experimental import pallas as pl
from jax.experimental.pallas import tpu as pltpu
from jax.experimental.pallas import tpu_sc as plsc

_NW = 32
_CW = 128
_TOK = 512


def _gelu(x):
    return 0.5 * x * (1.0 + lax.erf(x * 0.7071067811865476))


def _sc_gather(table, ids3):
    NW, CH, CW = ids3.shape
    d = table.shape[1]
    n = NW * CH * CW
    mesh = plsc.VectorSubcoreMesh(core_axis_name="c", subcore_axis_name="s")

    @functools.partial(
        pl.kernel,
        mesh=mesh,
        out_type=jax.ShapeDtypeStruct((n, d), jnp.float32),
        scratch_types=[
            pltpu.VMEM((CH, CW), jnp.int32),
            pltpu.VMEM((CW, d), jnp.float32),
            pltpu.VMEM((CW, d), jnp.float32),
            pltpu.SemaphoreType.DMA,
            pltpu.SemaphoreType.DMA,
        ],
    )
    def gather_kernel(table_hbm, idx_hbm, out_hbm, idx_v, buf0, buf1, sem0, sem1):
        wid = lax.axis_index("s") * 2 + lax.axis_index("c")
        pltpu.sync_copy(idx_hbm.at[wid], idx_v)
        bufs = (buf0, buf1)
        sems = (sem0, sem1)
        cps = [
            pltpu.async_copy(table_hbm.at[idx_v.at[0]], buf0, sem0),
            pltpu.async_copy(table_hbm.at[idx_v.at[1]], buf1, sem1),
        ]
        for c in range(CH):
            b = c % 2
            cps[b].wait()
            pltpu.sync_copy(bufs[b], out_hbm.at[pl.ds((wid * CH + c) * CW, CW)])
            if c + 2 < CH:
                cps[b] = pltpu.async_copy(table_hbm.at[idx_v.at[c + 2]], bufs[b], sems[b])

    return gather_kernel(table, ids3)


def _tc_body(actor_ref, role_ref, career_ref, genre_ref, rtab_ref,
             cW1_ref, cb1_ref, cW2_ref, cb2_ref, gW_ref, gb_ref,
             fW1a_ref, fW1b_ref, fW1c_ref, fW1g_ref, fb1_ref, fW2_ref, fb2_ref,
             out_ref):
    f32 = jnp.float32
    c1 = _gelu(jnp.dot(career_ref[...], cW1_ref[...], preferred_element_type=f32)
               + cb1_ref[...])
    career_emb = jnp.dot(c1, cW2_ref[...], preferred_element_type=f32) + cb2_ref[...]
    genre_emb = (jnp.dot(genre_ref[...], gW_ref[...], preferred_element_type=f32)
                 + gb_ref[...])
    h = jnp.dot(actor_ref[...], fW1a_ref[...], preferred_element_type=f32)
    h = h + jnp.dot(career_emb, fW1c_ref[...], preferred_element_type=f32)
    h = h + jnp.dot(genre_emb, fW1g_ref[...], preferred_element_type=f32)
    rT = jnp.dot(rtab_ref[...], fW1b_ref[...], preferred_element_type=f32)
    r = role_ref[...]
    for k in range(rtab_ref.shape[0]):
        h = h + jnp.where(r == k, 1.0, 0.0) * rT[k:k + 1, :]
    h = _gelu(h + fb1_ref[...])
    out_ref[...] = jnp.dot(h, fW2_ref[...], preferred_element_type=f32) + fb2_ref[...]


def _tc_fused(actor_emb, roles, career, genre, role_table, cW1, cb1, cW2, cb2,
              gW, gb, fW1a, fW1b, fW1c, fW1g, fb1, fW2, fb2):
    n, d = actor_emb.shape
    hdim = fW2.shape[1]
    T = _TOK

    def tok(i):
        return (i, 0)

    def full(a):
        return pl.BlockSpec(a.shape, lambda i: tuple(0 for _ in a.shape))

    in_specs = [
        pl.BlockSpec((T, d), tok),
        pl.BlockSpec((T, 1), tok),
        pl.BlockSpec((T, career.shape[1]), tok),
        pl.BlockSpec((T, genre.shape[1]), tok),
        full(role_table), full(cW1), full(cb1), full(cW2), full(cb2),
        full(gW), full(gb), full(fW1a), full(fW1b), full(fW1c), full(fW1g),
        full(fb1), full(fW2), full(fb2),
    ]
    return pl.pallas_call(
        _tc_body,
        grid=(n // T,),
        in_specs=in_specs,
        out_specs=pl.BlockSpec((T, hdim), tok),
        out_shape=jax.ShapeDtypeStruct((n, hdim), jnp.float32),
    )(actor_emb, roles, career, genre, role_table, cW1, cb1, cW2, cb2,
      gW, gb, fW1a, fW1b, fW1c, fW1g, fb1, fW2, fb2)


def kernel(actor_ids, role_types, career_features, genre_distribution,
           actor_table, role_table, cW1, cb1, cW2, cb2, gW, gb,
           fW1, fb1, fW2, fb2):
    bsz, seq = actor_ids.shape
    n = bsz * seq
    d = actor_table.shape[1]
    q = cW1.shape[1]
    ids3 = actor_ids.astype(jnp.int32).reshape(_NW, n // (_NW * _CW), _CW)
    actor_emb = _sc_gather(actor_table, ids3)
    fW1a = fW1[:d]
    fW1b = fW1[d:d + q]
    fW1c = fW1[d + q:d + 2 * q]
    fW1g = fW1[d + 2 * q:]
    out = _tc_fused(
        actor_emb,
        role_types.reshape(n, 1).astype(jnp.int32),
        career_features.reshape(n, -1),
        genre_distribution.reshape(n, -1),
        role_table, cW1, cb1.reshape(1, -1), cW2, cb2.reshape(1, -1),
        gW, gb.reshape(1, -1),
        fW1a, fW1b, fW1c, fW1g, fb1.reshape(1, -1), fW2, fb2.reshape(1, -1))
    return out.reshape(bsz, seq, -1)


# speedup vs baseline: 2.0299x; 1.2500x over previous
import functools

import jax
import jax.numpy as jnp
from jax import lax
from jax.experimental import pallas as pl
from jax.experimental.pallas import tpu as pltpu
from jax.experimental.pallas import tpu_sc as plsc

_NW = 32
_CW = 128
_BB = 128


def _gelu(x):
    return 0.5 * x * (1.0 + lax.erf(x * 0.7071067811865476))


def _sc_gather(table, ids3):
    NW, CH, CW = ids3.shape
    d = table.shape[1]
    n = NW * CH * CW
    mesh = plsc.VectorSubcoreMesh(core_axis_name="c", subcore_axis_name="s")

    @functools.partial(
        pl.kernel,
        mesh=mesh,
        out_type=jax.ShapeDtypeStruct((n, d), jnp.float32),
        scratch_types=[
            pltpu.VMEM((CH, CW), jnp.int32),
            pltpu.VMEM((CW, d), jnp.float32),
            pltpu.VMEM((CW, d), jnp.float32),
            pltpu.SemaphoreType.DMA,
            pltpu.SemaphoreType.DMA,
        ],
    )
    def gather_kernel(table_hbm, idx_hbm, out_hbm, idx_v, buf0, buf1, sem0, sem1):
        wid = lax.axis_index("s") * 2 + lax.axis_index("c")
        pltpu.sync_copy(idx_hbm.at[wid], idx_v)
        bufs = (buf0, buf1)
        sems = (sem0, sem1)
        cps = [
            pltpu.async_copy(table_hbm.at[idx_v.at[0]], buf0, sem0),
            pltpu.async_copy(table_hbm.at[idx_v.at[1]], buf1, sem1),
        ]
        for c in range(CH):
            b = c % 2
            cps[b].wait()
            pltpu.sync_copy(bufs[b], out_hbm.at[pl.ds((wid * CH + c) * CW, CW)])
            if c + 2 < CH:
                cps[b] = pltpu.async_copy(table_hbm.at[idx_v.at[c + 2]], bufs[b], sems[b])

    return gather_kernel(table, ids3)


def _tc_body(actor_ref, small_ref, rtab_ref,
             cW1_ref, cb1_ref, cW2_ref, cb2_ref, gW_ref, gb_ref,
             fW1a_ref, fW1b_ref, fW1c_ref, fW1g_ref, fb1_ref, fW2_ref, fb2_ref,
             out_ref):
    f32 = jnp.float32
    bb, seq, hdim = out_ref.shape
    t = bb * seq
    nc = cW1_ref.shape[0]
    ng = gW_ref.shape[0]
    small2 = small_ref[...].reshape(t, small_ref.shape[2])
    career2 = small2[:, :nc]
    genre2 = small2[:, nc:nc + ng]
    rf = small2[:, nc + ng:nc + ng + 1]
    c1 = _gelu(jnp.dot(career2, cW1_ref[...], preferred_element_type=f32)
               + cb1_ref[...])
    career_emb = jnp.dot(c1, cW2_ref[...], preferred_element_type=f32) + cb2_ref[...]
    genre_emb = (jnp.dot(genre2, gW_ref[...], preferred_element_type=f32)
                 + gb_ref[...])
    h = jnp.dot(actor_ref[...], fW1a_ref[...], preferred_element_type=f32)
    h = h + jnp.dot(career_emb, fW1c_ref[...], preferred_element_type=f32)
    h = h + jnp.dot(genre_emb, fW1g_ref[...], preferred_element_type=f32)
    rT = jnp.dot(rtab_ref[...], fW1b_ref[...], preferred_element_type=f32)
    for k in range(rtab_ref.shape[0]):
        h = h + jnp.where(rf == k, 1.0, 0.0) * rT[k:k + 1, :]
    h = _gelu(h + fb1_ref[...])
    out = jnp.dot(h, fW2_ref[...], preferred_element_type=f32) + fb2_ref[...]
    out_ref[...] = out.reshape(bb, seq, hdim)


def _tc_fused(actor_emb, small, role_table, cW1, cb1, cW2, cb2,
              gW, gb, fW1a, fW1b, fW1c, fW1g, fb1, fW2, fb2):
    bsz, seq, nsmall = small.shape
    d = actor_emb.shape[1]
    hdim = fW2.shape[1]
    bb = _BB

    def full(a):
        return pl.BlockSpec(a.shape, lambda i: tuple(0 for _ in a.shape))

    in_specs = [
        pl.BlockSpec((bb * seq, d), lambda i: (i, 0)),
        pl.BlockSpec((bb, seq, nsmall), lambda i: (i, 0, 0)),
        full(role_table), full(cW1), full(cb1), full(cW2), full(cb2),
        full(gW), full(gb), full(fW1a), full(fW1b), full(fW1c), full(fW1g),
        full(fb1), full(fW2), full(fb2),
    ]
    return pl.pallas_call(
        _tc_body,
        grid=(bsz // bb,),
        in_specs=in_specs,
        out_specs=pl.BlockSpec((bb, seq, hdim), lambda i: (i, 0, 0)),
        out_shape=jax.ShapeDtypeStruct((bsz, seq, hdim), jnp.float32),
    )(actor_emb, small, role_table, cW1, cb1, cW2, cb2,
      gW, gb, fW1a, fW1b, fW1c, fW1g, fb1, fW2, fb2)


def kernel(actor_ids, role_types, career_features, genre_distribution,
           actor_table, role_table, cW1, cb1, cW2, cb2, gW, gb,
           fW1, fb1, fW2, fb2):
    bsz, seq = actor_ids.shape
    n = bsz * seq
    d = actor_table.shape[1]
    q = cW1.shape[1]
    ids3 = actor_ids.astype(jnp.int32).reshape(_NW, n // (_NW * _CW), _CW)
    actor_emb = _sc_gather(actor_table, ids3)
    fW1a = fW1[:d]
    fW1b = fW1[d:d + q]
    fW1c = fW1[d + q:d + 2 * q]
    fW1g = fW1[d + 2 * q:]
    small = jnp.concatenate(
        [career_features, genre_distribution,
         role_types[..., None].astype(jnp.float32)], axis=-1)
    out = _tc_fused(
        actor_emb, small,
        role_table, cW1, cb1.reshape(1, -1), cW2, cb2.reshape(1, -1),
        gW, gb.reshape(1, -1),
        fW1a, fW1b, fW1c, fW1g, fb1.reshape(1, -1), fW2, fb2.reshape(1, -1))
    return out
